# baseline (speedup 1.0000x reference)
import functools

import jax
import jax.numpy as jnp
from jax import lax
from jax.experimental import pallas as pl
from jax.experimental.pallas import tpu as pltpu
from jax.experimental.pallas import tpu_sc as plsc

_B, _T, _P, _F = 16, 50, 200, 128
_N = _B * _T
_ROW = _P * _F
_NW = 32
_S = _N // _NW
_NV = _F // 16


def _sc_body(x_hbm, tl_hbm, tt_hbm, pt_hbm, ft_hbm, out_hbm,
             buf0, buf1, ptv, ftv, ttv, tlv,
             lsem0, lsem1, ssem0, ssem1):
    cid = lax.axis_index("c")
    sid = lax.axis_index("s")
    wid = sid * 2 + cid
    base = wid * _S

    pltpu.sync_copy(pt_hbm, ptv)
    pltpu.sync_copy(ft_hbm, ftv)
    pltpu.sync_copy(tt_hbm, ttv)
    pltpu.sync_copy(tl_hbm, tlv)

    bufs = [buf0, buf1]
    lsems = [lsem0, lsem1]
    ssems = [ssem0, ssem1]
    loads = [None, None]
    stores = [None, None]

    f_vecs = [ftv[pl.ds(k * 16, 16)] for k in range(_NV)]

    loads[0] = pltpu.async_copy(x_hbm.at[base], bufs[0], lsems[0])

    for i in range(_S):
        b = i % 2
        nb = (i + 1) % 2
        if i + 1 < _S:
            if i >= 1:
                stores[nb].wait()
            loads[nb] = pltpu.async_copy(
                x_hbm.at[base + i + 1], bufs[nb], lsems[nb])
        loads[b].wait()

        svec = jnp.full((16,), base + i, dtype=jnp.int32)
        tl16 = plsc.load_gather(tlv, [svec])
        te = plsc.load_gather(ttv, [tl16 // 3])

        buf = bufs[b]

        def p_body(p, carry, buf=buf, te=te):
            pvec = jnp.full((16,), p, dtype=jnp.int32)
            pte = plsc.load_gather(ptv, [pvec]) + te
            for k in range(_NV):
                off = p * _F + k * 16
                buf[pl.ds(off, 16)] = buf[pl.ds(off, 16)] + (f_vecs[k] + pte)
            return carry

        lax.fori_loop(0, _P, p_body, 0)

        stores[b] = pltpu.async_copy(buf, out_hbm.at[base + i], ssems[b])

    stores[(_S - 2) % 2].wait()
    stores[(_S - 1) % 2].wait()


_sc_call = functools.partial(
    pl.kernel,
    mesh=plsc.VectorSubcoreMesh(core_axis_name="c", subcore_axis_name="s"),
    compiler_params=pltpu.CompilerParams(needs_layout_passes=False),
    out_type=jax.ShapeDtypeStruct((_N, _ROW), jnp.float32),
    scratch_types=[
        pltpu.VMEM((_ROW,), jnp.float32),
        pltpu.VMEM((_ROW,), jnp.float32),
        pltpu.VMEM((_P,), jnp.float32),
        pltpu.VMEM((_F,), jnp.float32),
        pltpu.VMEM((8,), jnp.float32),
        pltpu.VMEM((_N,), jnp.int32),
        pltpu.SemaphoreType.DMA,
        pltpu.SemaphoreType.DMA,
        pltpu.SemaphoreType.DMA,
        pltpu.SemaphoreType.DMA,
    ],
)(_sc_body)


@jax.jit
def kernel(x, time_list, time_table, point_table, f_table):
    x2 = x.reshape(_N, _ROW)
    tl = time_list.reshape(_N).astype(jnp.int32)
    out = _sc_call(x2, tl, time_table.reshape(-1), point_table.reshape(-1),
                   f_table.reshape(-1))
    return out.reshape(_B, _T, _P, _F)


# speedup vs baseline: 2.1062x; 2.1062x over previous
import functools

import jax
import jax.numpy as jnp
from jax import lax
from jax.experimental import pallas as pl
from jax.experimental.pallas import tpu as pltpu
from jax.experimental.pallas import tpu_sc as plsc

_B, _T, _P, _F = 16, 50, 200, 128
_N = _B * _T
_NW = 32
_S = _N // _NW
_NV = _F // 16


def _sc_body(x_hbm, tl_hbm, tt_hbm, pt_hbm, ft_hbm, out_hbm,
             buf0, buf1, ptv, ftv, ttv, tlv,
             lsem0, lsem1, ssem0, ssem1):
    cid = lax.axis_index("c")
    sid = lax.axis_index("s")
    wid = sid * 2 + cid
    base = wid * _S

    pltpu.sync_copy(pt_hbm, ptv)
    pltpu.sync_copy(ft_hbm, ftv)
    pltpu.sync_copy(tt_hbm, ttv)
    pltpu.sync_copy(tl_hbm, tlv)

    bufs = [buf0, buf1]
    lsems = [lsem0, lsem1]
    ssems = [ssem0, ssem1]
    loads = [None, None]
    stores = [None, None]

    zeros = jnp.zeros((16,), jnp.int32)
    lane = lax.iota(jnp.int32, 16)
    f_vecs = [plsc.load_gather(ftv, [lane + 16 * k, zeros])
              for k in range(_NV)]

    loads[0] = pltpu.async_copy(x_hbm.at[base], bufs[0], lsems[0])

    for i in range(_S):
        b = i % 2
        nb = (i + 1) % 2
        if i + 1 < _S:
            if i >= 1:
                stores[nb].wait()
            loads[nb] = pltpu.async_copy(
                x_hbm.at[base + i + 1], bufs[nb], lsems[nb])
        loads[b].wait()

        s = base + i
        rvec = jnp.full((16,), s // _T, dtype=jnp.int32)
        cvec = jnp.full((16,), s % _T, dtype=jnp.int32)
        tl16 = plsc.load_gather(tlv, [rvec, cvec])
        te = plsc.load_gather(ttv, [tl16 // 3, zeros])

        buf = bufs[b]

        def p_body(p, carry, buf=buf, te=te):
            pvec = jnp.full((16,), p, dtype=jnp.int32)
            pte = plsc.load_gather(ptv, [pvec, zeros]) + te
            for k in range(_NV):
                col = k * 16
                buf[p, pl.ds(col, 16)] = (
                    buf[p, pl.ds(col, 16)] + (f_vecs[k] + pte))
            return carry

        lax.fori_loop(0, _P, p_body, 0)

        stores[b] = pltpu.async_copy(buf, out_hbm.at[s], ssems[b])

    stores[(_S - 2) % 2].wait()
    stores[(_S - 1) % 2].wait()


_sc_call = functools.partial(
    pl.kernel,
    mesh=plsc.VectorSubcoreMesh(core_axis_name="c", subcore_axis_name="s"),
    compiler_params=pltpu.CompilerParams(needs_layout_passes=False),
    out_type=jax.ShapeDtypeStruct((_N, _P, _F), jnp.float32),
    scratch_types=[
        pltpu.VMEM((_P, _F), jnp.float32),
        pltpu.VMEM((_P, _F), jnp.float32),
        pltpu.VMEM((_P, 1), jnp.float32),
        pltpu.VMEM((_F, 1), jnp.float32),
        pltpu.VMEM((8, 1), jnp.float32),
        pltpu.VMEM((_B, _T), jnp.int32),
        pltpu.SemaphoreType.DMA,
        pltpu.SemaphoreType.DMA,
        pltpu.SemaphoreType.DMA,
        pltpu.SemaphoreType.DMA,
    ],
)(_sc_body)


@jax.jit
def kernel(x, time_list, time_table, point_table, f_table):
    x3 = x.reshape(_N, _P, _F)
    tl = time_list.astype(jnp.int32)
    out = _sc_call(x3, tl, time_table, point_table, f_table)
    return out.reshape(_B, _T, _P, _F)


# speedup vs baseline: 2.1722x; 1.0313x over previous
import functools

import jax
import jax.numpy as jnp
from jax import lax
from jax.experimental import pallas as pl
from jax.experimental.pallas import tpu as pltpu
from jax.experimental.pallas import tpu_sc as plsc

_B, _T, _P, _F = 16, 50, 200, 128
_N = _B * _T
_NW = 32
_S = _N // _NW
_NV = _F // 16
_NBUF = 4


def _sc_body(x_hbm, tl_hbm, tt_hbm, pt_hbm, ft_hbm, out_hbm,
             buf0, buf1, buf2, buf3, ptv, ftv, ttv, tlv,
             lsem0, lsem1, lsem2, lsem3, ssem0, ssem1, ssem2, ssem3):
    cid = lax.axis_index("c")
    sid = lax.axis_index("s")
    wid = sid * 2 + cid
    base = wid * _S

    pltpu.sync_copy(pt_hbm, ptv)
    pltpu.sync_copy(ft_hbm, ftv)
    pltpu.sync_copy(tt_hbm, ttv)
    pltpu.sync_copy(tl_hbm, tlv)

    bufs = [buf0, buf1, buf2, buf3]
    lsems = [lsem0, lsem1, lsem2, lsem3]
    ssems = [ssem0, ssem1, ssem2, ssem3]
    loads = [None] * _NBUF
    stores = [None] * _NBUF

    zeros = jnp.zeros((16,), jnp.int32)
    lane = lax.iota(jnp.int32, 16)
    f_vecs = [plsc.load_gather(ftv, [lane + 16 * k, zeros])
              for k in range(_NV)]

    for j in range(_NBUF - 1):
        loads[j] = pltpu.async_copy(x_hbm.at[base + j], bufs[j], lsems[j])

    for i in range(_S):
        b = i % _NBUF
        nb = (i + _NBUF - 1) % _NBUF
        if i + _NBUF - 1 < _S:
            if stores[nb] is not None:
                stores[nb].wait()
            loads[nb] = pltpu.async_copy(
                x_hbm.at[base + i + _NBUF - 1], bufs[nb], lsems[nb])
        loads[b].wait()

        s = base + i
        rvec = jnp.full((16,), s // _T, dtype=jnp.int32)
        cvec = jnp.full((16,), s % _T, dtype=jnp.int32)
        tl16 = plsc.load_gather(tlv, [rvec, cvec])
        te = plsc.load_gather(ttv, [zeros, tl16 // 3])

        buf = bufs[b]

        def p_body(p, carry, buf=buf, te=te):
            pvec = jnp.full((16,), p, dtype=jnp.int32)
            pte = plsc.load_gather(ptv, [pvec // 100, pvec % 100]) + te
            for k in range(_NV):
                col = k * 16
                buf[p, pl.ds(col, 16)] = (
                    buf[p, pl.ds(col, 16)] + (f_vecs[k] + pte))
            return carry

        lax.fori_loop(0, _P, p_body, 0)

        stores[b] = pltpu.async_copy(buf, out_hbm.at[s], ssems[b])

    for j in range(max(0, _S - _NBUF), _S):
        stores[j % _NBUF].wait()


_sc_call = functools.partial(
    pl.kernel,
    mesh=plsc.VectorSubcoreMesh(core_axis_name="c", subcore_axis_name="s"),
    compiler_params=pltpu.CompilerParams(needs_layout_passes=False),
    out_type=jax.ShapeDtypeStruct((_N, _P, _F), jnp.float32),
    scratch_types=[
        pltpu.VMEM((_P, _F), jnp.float32),
        pltpu.VMEM((_P, _F), jnp.float32),
        pltpu.VMEM((_P, _F), jnp.float32),
        pltpu.VMEM((_P, _F), jnp.float32),
        pltpu.VMEM((2, 100), jnp.float32),
        pltpu.VMEM((1, _F), jnp.float32),
        pltpu.VMEM((1, 8), jnp.float32),
        pltpu.VMEM((_B, _T), jnp.int32),
        pltpu.SemaphoreType.DMA,
        pltpu.SemaphoreType.DMA,
        pltpu.SemaphoreType.DMA,
        pltpu.SemaphoreType.DMA,
        pltpu.SemaphoreType.DMA,
        pltpu.SemaphoreType.DMA,
        pltpu.SemaphoreType.DMA,
        pltpu.SemaphoreType.DMA,
    ],
)(_sc_body)


@jax.jit
def kernel(x, time_list, time_table, point_table, f_table):
    x3 = x.reshape(_N, _P, _F)
    tl = time_list.astype(jnp.int32)
    out = _sc_call(x3, tl, time_table.reshape(1, 8),
                   point_table.reshape(2, 100), f_table.reshape(1, _F))
    return out.reshape(_B, _T, _P, _F)


# speedup vs baseline: 2.1756x; 1.0016x over previous
import functools

import jax
import jax.numpy as jnp
from jax import lax
from jax.experimental import pallas as pl
from jax.experimental.pallas import tpu as pltpu
from jax.experimental.pallas import tpu_sc as plsc

_B, _T, _P, _F = 16, 50, 200, 128
_N = _B * _T
_NW = 32
_S = _N // _NW
_NV = _F // 16
_NBUF = 4


def _sc_body(x_hbm, tl_hbm, tt_hbm, pt_hbm, ft_hbm, out_hbm,
             buf0, buf1, buf2, buf3, ptv, ftv, ttv, tlv,
             lsem0, lsem1, lsem2, lsem3, ssem0, ssem1, ssem2, ssem3):
    cid = lax.axis_index("c")
    sid = lax.axis_index("s")
    wid = sid * 2 + cid
    base = wid * _S

    pltpu.sync_copy(pt_hbm, ptv)
    pltpu.sync_copy(ft_hbm, ftv)
    pltpu.sync_copy(tt_hbm, ttv)
    pltpu.sync_copy(tl_hbm, tlv)

    bufs = [buf0, buf1, buf2, buf3]
    lsems = [lsem0, lsem1, lsem2, lsem3]
    ssems = [ssem0, ssem1, ssem2, ssem3]
    loads = [None] * _NBUF
    stores = [None] * _NBUF

    zeros = jnp.zeros((16,), jnp.int32)
    lane = lax.iota(jnp.int32, 16)
    f_vecs = [plsc.load_gather(ftv, [zeros, lane + 16 * k])
              for k in range(_NV)]

    for j in range(_NBUF - 1):
        loads[j] = pltpu.async_copy(x_hbm.at[base + j], bufs[j], lsems[j])

    for i in range(_S):
        b = i % _NBUF
        nb = (i + _NBUF - 1) % _NBUF
        if i + _NBUF - 1 < _S:
            if stores[nb] is not None:
                stores[nb].wait()
            loads[nb] = pltpu.async_copy(
                x_hbm.at[base + i + _NBUF - 1], bufs[nb], lsems[nb])
        loads[b].wait()

        s = base + i
        rvec = jnp.full((16,), s // _T, dtype=jnp.int32)
        cvec = jnp.full((16,), s % _T, dtype=jnp.int32)
        tl16 = plsc.load_gather(tlv, [rvec, cvec])
        te = plsc.load_gather(ttv, [zeros, tl16 // 3])

        buf = bufs[b]

        def p_body(p, carry, buf=buf, te=te):
            pvec = jnp.full((16,), p, dtype=jnp.int32)
            pte = plsc.load_gather(ptv, [pvec // 100, pvec % 100]) + te
            for k in range(_NV):
                col = k * 16
                buf[p, pl.ds(col, 16)] = (
                    buf[p, pl.ds(col, 16)] + (f_vecs[k] + pte))
            return carry

        lax.fori_loop(0, _P, p_body, 0)

        stores[b] = pltpu.async_copy(buf, out_hbm.at[s], ssems[b])

    for j in range(max(0, _S - _NBUF), _S):
        stores[j % _NBUF].wait()


_sc_call = functools.partial(
    pl.kernel,
    mesh=plsc.VectorSubcoreMesh(core_axis_name="c", subcore_axis_name="s"),
    compiler_params=pltpu.CompilerParams(needs_layout_passes=False),
    out_type=jax.ShapeDtypeStruct((_N, _P, _F), jnp.float32),
    scratch_types=[
        pltpu.VMEM((_P, _F), jnp.float32),
        pltpu.VMEM((_P, _F), jnp.float32),
        pltpu.VMEM((_P, _F), jnp.float32),
        pltpu.VMEM((_P, _F), jnp.float32),
        pltpu.VMEM((2, 100), jnp.float32),
        pltpu.VMEM((1, _F), jnp.float32),
        pltpu.VMEM((1, 8), jnp.float32),
        pltpu.VMEM((_B, _T), jnp.int32),
        pltpu.SemaphoreType.DMA,
        pltpu.SemaphoreType.DMA,
        pltpu.SemaphoreType.DMA,
        pltpu.SemaphoreType.DMA,
        pltpu.SemaphoreType.DMA,
        pltpu.SemaphoreType.DMA,
        pltpu.SemaphoreType.DMA,
        pltpu.SemaphoreType.DMA,
    ],
)(_sc_body)


@jax.jit
def kernel(x, time_list, time_table, point_table, f_table):
    x3 = x.reshape(_N, _P, _F)
    tl = time_list.astype(jnp.int32)
    out = _sc_call(x3, tl, time_table.reshape(1, 8),
                   point_table.reshape(2, 100), f_table.reshape(1, _F))
    return out.reshape(_B, _T, _P, _F)


# speedup vs baseline: 4.6546x; 2.1395x over previous
import functools

import jax
import jax.numpy as jnp
from jax.experimental import pallas as pl
from jax.experimental.pallas import tpu as pltpu

_B, _T, _P, _F = 16, 50, 200, 128
_N = _B * _T
_G = 100


def _tc_body(tl_sp, tt_sp, x_ref, pt_ref, ft_ref, o_ref):
    g = pl.program_id(0)
    pf = pt_ref[...] + ft_ref[...]
    for r in range(_G):
        idx = tl_sp[g * _G + r] // 3
        te = tt_sp[idx]
        o_ref[r] = x_ref[r] + (pf + te)


@jax.jit
def kernel(x, time_list, time_table, point_table, f_table):
    x3 = x.reshape(_N, _P, _F)
    tl = time_list.reshape(_N).astype(jnp.int32)
    tt = time_table.reshape(-1)
    pt = point_table.reshape(_P, 1)
    ft = f_table.reshape(1, _F)

    grid_spec = pltpu.PrefetchScalarGridSpec(
        num_scalar_prefetch=2,
        grid=(_N // _G,),
        in_specs=[
            pl.BlockSpec((_G, _P, _F), lambda g, tl_sp, tt_sp: (g, 0, 0)),
            pl.BlockSpec((_P, 1), lambda g, tl_sp, tt_sp: (0, 0)),
            pl.BlockSpec((1, _F), lambda g, tl_sp, tt_sp: (0, 0)),
        ],
        out_specs=pl.BlockSpec((_G, _P, _F), lambda g, tl_sp, tt_sp: (g, 0, 0)),
    )
    out = pl.pallas_call(
        _tc_body,
        grid_spec=grid_spec,
        out_shape=jax.ShapeDtypeStruct((_N, _P, _F), jnp.float32),
    )(tl, tt, x3, pt, ft)
    return out.reshape(_B, _T, _P, _F)
